# baseline (speedup 1.0000x reference)
import functools

import jax
import jax.numpy as jnp
from jax import lax
from jax.experimental import pallas as pl
from jax.experimental.pallas import tpu as pltpu
from jax.experimental.pallas import tpu_sc as plsc

N_CHUNKS = 32
G = N_CHUNKS + 1
IN_DIM = 256
OUT_DIM = 128
BATCH = 1024
P = IN_DIM // 2
ROWS = P * G * G

NC, NS, LANES = 2, 16, 16
NW = NC * NS
TPW = BATCH // NW
K_PER_TOK = 4 * P
CHUNK = 128
CPT = K_PER_TOK // CHUNK
OVR = OUT_DIM // LANES


def _linear_body(x_ref, w_ref, b_ref, flag_ref, out_ref):
    acc = jnp.dot(w_ref[...], x_ref[...], preferred_element_type=jnp.float32)
    acc = acc + b_ref[...]
    acc = jnp.where(flag_ref[0] != 0, jnp.maximum(acc, 0.0), acc)
    out_ref[...] = acc


def _prep_body(xa_ref, xb_ref, sa_ref, sb_ref, ba_ref, bb_ref, ws_ref,
               idx_ref, w_ref):
    xa = jnp.tanh(xa_ref[...] * sa_ref[...] + ba_ref[...])
    xb = jnp.tanh(xb_ref[...] * sb_ref[...] + bb_ref[...])
    ua = jnp.clip((xa + 1.0) * 0.5 * N_CHUNKS, 0.0, N_CHUNKS - 1e-4)
    ub = jnp.clip((xb + 1.0) * 0.5 * N_CHUNKS, 0.0, N_CHUNKS - 1e-4)
    ia = jnp.floor(ua)
    ib = jnp.floor(ub)
    fa = ua - ia
    fb = ub - ib
    ia_i = ia.astype(jnp.int32)
    ib_i = ib.astype(jnp.int32)
    pvec = lax.broadcasted_iota(jnp.int32, (BATCH, P), 1)
    base = pvec * (G * G) + ia_i * G + ib_i
    wl = ws_ref[0]
    idx_ref[:, 0 * P:1 * P] = base
    idx_ref[:, 1 * P:2 * P] = base + G
    idx_ref[:, 2 * P:3 * P] = base + 1
    idx_ref[:, 3 * P:4 * P] = base + (G + 1)
    w_ref[:, 0 * P:1 * P] = (1.0 - fa) * (1.0 - fb) * wl
    w_ref[:, 1 * P:2 * P] = fa * (1.0 - fb) * wl
    w_ref[:, 2 * P:3 * P] = (1.0 - fa) * fb * wl
    w_ref[:, 3 * P:4 * P] = fa * fb * wl


_prep_call = pl.pallas_call(
    _prep_body,
    out_shape=(
        jax.ShapeDtypeStruct((BATCH, K_PER_TOK), jnp.int32),
        jax.ShapeDtypeStruct((BATCH, K_PER_TOK), jnp.float32),
    ),
    in_specs=[
        pl.BlockSpec(memory_space=pltpu.VMEM),
        pl.BlockSpec(memory_space=pltpu.VMEM),
        pl.BlockSpec(memory_space=pltpu.VMEM),
        pl.BlockSpec(memory_space=pltpu.VMEM),
        pl.BlockSpec(memory_space=pltpu.VMEM),
        pl.BlockSpec(memory_space=pltpu.VMEM),
        pl.BlockSpec(memory_space=pltpu.SMEM),
    ],
)

_linear_call = pl.pallas_call(
    _linear_body,
    out_shape=jax.ShapeDtypeStruct((OUT_DIM, BATCH), jnp.float32),
    in_specs=[
        pl.BlockSpec(memory_space=pltpu.VMEM),
        pl.BlockSpec(memory_space=pltpu.VMEM),
        pl.BlockSpec(memory_space=pltpu.VMEM),
        pl.BlockSpec(memory_space=pltpu.SMEM),
    ],
)


def _sc_body(table_hbm, idx_hbm, w_hbm, out_hbm, idx_v, w_v, rows_v, out_v, sem):
    wid = lax.axis_index("s") * NC + lax.axis_index("c")
    rbase = wid * (CPT * TPW)
    pltpu.sync_copy(idx_hbm.at[pl.ds(rbase, CPT * TPW)], idx_v)
    pltpu.sync_copy(w_hbm.at[pl.ds(rbase, CPT * TPW)], w_v)

    def token_body(tok, carry):
        acc = tuple(jnp.zeros((LANES,), jnp.float32) for _ in range(OVR))
        for c in range(CPT):
            j = tok * CPT + c
            pltpu.async_copy(table_hbm.at[idx_v.at[j]], rows_v, sem).wait()

            def g_body(g, acc, j=j):
                acc = list(acc)
                off = pl.multiple_of(g * LANES, LANES)
                wv = w_v[j, pl.ds(off, LANES)]
                for r in range(LANES):
                    ws = jnp.broadcast_to(wv[r], (LANES,))
                    row = g * LANES + r
                    for ov in range(OVR):
                        acc[ov] = acc[ov] + ws * rows_v[row, pl.ds(ov * LANES, LANES)]
                return tuple(acc)

            acc = lax.fori_loop(0, CHUNK // LANES, g_body, acc)
        for ov in range(OVR):
            out_v[tok, pl.ds(ov * LANES, LANES)] = acc[ov]
        return carry

    lax.fori_loop(0, TPW, token_body, jnp.int32(0))
    pltpu.sync_copy(out_v, out_hbm.at[pl.ds(wid * TPW, TPW)])


@functools.cache
def _sc_gather_call():
    return pl.kernel(
        _sc_body,
        out_type=jax.ShapeDtypeStruct((BATCH, OUT_DIM), jnp.float32),
        mesh=plsc.VectorSubcoreMesh(core_axis_name="c", subcore_axis_name="s"),
        scratch_types=[
            pltpu.VMEM((CPT * TPW, CHUNK), jnp.int32),
            pltpu.VMEM((CPT * TPW, CHUNK), jnp.float32),
            pltpu.VMEM((CHUNK, OUT_DIM), jnp.float32),
            pltpu.VMEM((TPW, OUT_DIM), jnp.float32),
            pltpu.SemaphoreType.DMA,
        ],
    )


def kernel(x, weight_lmKAN, apply_relu_linear, func_parameter, scale_parameters, bias_parameters, W_linear, bias_linear):
    flag = jnp.asarray(apply_relu_linear, jnp.int32).reshape((1,))
    wl = jnp.asarray(weight_lmKAN, jnp.float32).reshape((1,))

    linear_out = _linear_call(x, W_linear, bias_linear.reshape(OUT_DIM, 1), flag)

    xT = x.T
    idx, w = _prep_call(
        xT[:, 0::2], xT[:, 1::2],
        scale_parameters[0::2].reshape(1, P), scale_parameters[1::2].reshape(1, P),
        bias_parameters[0::2].reshape(1, P), bias_parameters[1::2].reshape(1, P),
        wl,
    )

    table = jnp.transpose(func_parameter, (3, 0, 1, 2)).reshape(ROWS, OUT_DIM)
    out_sc = _sc_gather_call()(table, idx.reshape(CPT * BATCH, CHUNK),
                               w.reshape(CPT * BATCH, CHUNK))
    return linear_out + out_sc.T


# speedup vs baseline: 1.3941x; 1.3941x over previous
import functools

import jax
import jax.numpy as jnp
from jax import lax
from jax.experimental import pallas as pl
from jax.experimental.pallas import tpu as pltpu
from jax.experimental.pallas import tpu_sc as plsc

N_CHUNKS = 32
G = N_CHUNKS + 1
IN_DIM = 256
OUT_DIM = 128
BATCH = 1024
P = IN_DIM // 2
ROWS = P * G * G

NC, NS, LANES = 2, 16, 16
NW = NC * NS
TPW = BATCH // NW
K_PER_TOK = 4 * P
CHUNK = 128
CPT = K_PER_TOK // CHUNK
OVR = OUT_DIM // LANES


def _linear_body(x_ref, w_ref, b_ref, flag_ref, out_ref):
    acc = jnp.dot(w_ref[...], x_ref[...], preferred_element_type=jnp.float32)
    acc = acc + b_ref[...]
    acc = jnp.where(flag_ref[0] != 0, jnp.maximum(acc, 0.0), acc)
    out_ref[...] = acc


def _prep_body(xa_ref, xb_ref, sa_ref, sb_ref, ba_ref, bb_ref, ws_ref,
               idx_ref, w_ref):
    xa = jnp.tanh(xa_ref[...] * sa_ref[...] + ba_ref[...])
    xb = jnp.tanh(xb_ref[...] * sb_ref[...] + bb_ref[...])
    ua = jnp.clip((xa + 1.0) * 0.5 * N_CHUNKS, 0.0, N_CHUNKS - 1e-4)
    ub = jnp.clip((xb + 1.0) * 0.5 * N_CHUNKS, 0.0, N_CHUNKS - 1e-4)
    ia = jnp.floor(ua)
    ib = jnp.floor(ub)
    fa = ua - ia
    fb = ub - ib
    ia_i = ia.astype(jnp.int32)
    ib_i = ib.astype(jnp.int32)
    pvec = lax.broadcasted_iota(jnp.int32, (BATCH, P), 1)
    base = pvec * (G * G) + ia_i * G + ib_i
    wl = ws_ref[0]
    idx_ref[:, 0 * P:1 * P] = base
    idx_ref[:, 1 * P:2 * P] = base + G
    idx_ref[:, 2 * P:3 * P] = base + 1
    idx_ref[:, 3 * P:4 * P] = base + (G + 1)
    w_ref[:, 0 * P:1 * P] = (1.0 - fa) * (1.0 - fb) * wl
    w_ref[:, 1 * P:2 * P] = fa * (1.0 - fb) * wl
    w_ref[:, 2 * P:3 * P] = (1.0 - fa) * fb * wl
    w_ref[:, 3 * P:4 * P] = fa * fb * wl


_prep_call = pl.pallas_call(
    _prep_body,
    out_shape=(
        jax.ShapeDtypeStruct((BATCH, K_PER_TOK), jnp.int32),
        jax.ShapeDtypeStruct((BATCH, K_PER_TOK), jnp.float32),
    ),
    in_specs=[
        pl.BlockSpec(memory_space=pltpu.VMEM),
        pl.BlockSpec(memory_space=pltpu.VMEM),
        pl.BlockSpec(memory_space=pltpu.VMEM),
        pl.BlockSpec(memory_space=pltpu.VMEM),
        pl.BlockSpec(memory_space=pltpu.VMEM),
        pl.BlockSpec(memory_space=pltpu.VMEM),
        pl.BlockSpec(memory_space=pltpu.SMEM),
    ],
)

_linear_call = pl.pallas_call(
    _linear_body,
    out_shape=jax.ShapeDtypeStruct((OUT_DIM, BATCH), jnp.float32),
    in_specs=[
        pl.BlockSpec(memory_space=pltpu.VMEM),
        pl.BlockSpec(memory_space=pltpu.VMEM),
        pl.BlockSpec(memory_space=pltpu.VMEM),
        pl.BlockSpec(memory_space=pltpu.SMEM),
    ],
)


def _sc_body(table_hbm, idx_hbm, w_hbm, out_hbm, idx_v, w_v, rows_v, out_v,
             sem0, sem1):
    wid = lax.axis_index("s") * NC + lax.axis_index("c")
    rbase = wid * (CPT * TPW)
    pltpu.sync_copy(idx_hbm.at[pl.ds(rbase, CPT * TPW)], idx_v)
    pltpu.sync_copy(w_hbm.at[pl.ds(rbase, CPT * TPW)], w_v)
    sems = (sem0, sem1)

    pltpu.async_copy(table_hbm.at[idx_v.at[0]], rows_v.at[0], sems[0])

    def token_body(tok, carry):
        acc = tuple(jnp.zeros((LANES,), jnp.float32) for _ in range(OVR))
        for c in range(CPT):
            j = tok * CPT + c
            buf = c & 1
            nxt = j + 1

            @pl.when(nxt < CPT * TPW)
            def _():
                pltpu.async_copy(table_hbm.at[idx_v.at[nxt]],
                                 rows_v.at[1 - buf], sems[1 - buf])

            pltpu.make_async_copy(table_hbm.at[idx_v.at[j]],
                                  rows_v.at[buf], sems[buf]).wait()

            def g_body(g, acc, j=j, buf=buf):
                acc = list(acc)
                off = pl.multiple_of(g * LANES, LANES)
                wv = w_v[j, pl.ds(off, LANES)]
                for r in range(LANES):
                    ws = jnp.broadcast_to(wv[r], (LANES,))
                    row = g * LANES + r
                    for ov in range(OVR):
                        acc[ov] = acc[ov] + ws * rows_v[buf, row,
                                                        pl.ds(ov * LANES, LANES)]
                return tuple(acc)

            acc = lax.fori_loop(0, CHUNK // LANES, g_body, acc)
        for ov in range(OVR):
            out_v[tok, pl.ds(ov * LANES, LANES)] = acc[ov]
        return carry

    lax.fori_loop(0, TPW, token_body, jnp.int32(0))
    pltpu.sync_copy(out_v, out_hbm.at[pl.ds(wid * TPW, TPW)])


@functools.cache
def _sc_gather_call():
    return pl.kernel(
        _sc_body,
        out_type=jax.ShapeDtypeStruct((BATCH, OUT_DIM), jnp.float32),
        mesh=plsc.VectorSubcoreMesh(core_axis_name="c", subcore_axis_name="s"),
        scratch_types=[
            pltpu.VMEM((CPT * TPW, CHUNK), jnp.int32),
            pltpu.VMEM((CPT * TPW, CHUNK), jnp.float32),
            pltpu.VMEM((2, CHUNK, OUT_DIM), jnp.float32),
            pltpu.VMEM((TPW, OUT_DIM), jnp.float32),
            pltpu.SemaphoreType.DMA,
            pltpu.SemaphoreType.DMA,
        ],
    )


def kernel(x, weight_lmKAN, apply_relu_linear, func_parameter, scale_parameters, bias_parameters, W_linear, bias_linear):
    flag = jnp.asarray(apply_relu_linear, jnp.int32).reshape((1,))
    wl = jnp.asarray(weight_lmKAN, jnp.float32).reshape((1,))

    linear_out = _linear_call(x, W_linear, bias_linear.reshape(OUT_DIM, 1), flag)

    xT = x.T
    idx, w = _prep_call(
        xT[:, 0::2], xT[:, 1::2],
        scale_parameters[0::2].reshape(1, P), scale_parameters[1::2].reshape(1, P),
        bias_parameters[0::2].reshape(1, P), bias_parameters[1::2].reshape(1, P),
        wl,
    )

    table = jnp.transpose(func_parameter, (3, 0, 1, 2)).reshape(ROWS, OUT_DIM)
    out_sc = _sc_gather_call()(table, idx.reshape(CPT * BATCH, CHUNK),
                               w.reshape(CPT * BATCH, CHUNK))
    return linear_out + out_sc.T
